# baseline (speedup 1.0000x reference)
import functools

import jax
import jax.numpy as jnp
from jax import lax
from jax.experimental import pallas as pl
from jax.experimental.pallas import tpu as pltpu
from jax.experimental.pallas import tpu_sc as plsc

L = 1025
NH = 16
NREL = 3972
NW = 32
GPW = 4
NCH = (L - 1) // 16
CTAIL = L - 16


_mesh = plsc.VectorSubcoreMesh(core_axis_name="c", subcore_axis_name="s")


@functools.partial(
    pl.kernel,
    mesh=_mesh,
    out_type=(
        jax.ShapeDtypeStruct((NH, L, L), jnp.float32),
        jax.ShapeDtypeStruct((NH, L), jnp.float32),
    ),
    scratch_types=[
        pltpu.VMEM((NREL * NH,), jnp.float32),
        pltpu.VMEM((8, L), jnp.int32),
        pltpu.VMEM((2, 8, L), jnp.float32),
        pltpu.VMEM((2, 8, L), jnp.float32),
        pltpu.VMEM((NH, L), jnp.float32),
        pltpu.SemaphoreType.DMA,
        pltpu.SemaphoreType.DMA,
    ],
    compiler_params=pltpu.CompilerParams(needs_layout_passes=False),
)
def _gather_bias(tab_hbm, idx_hbm, out_hbm, out2_hbm,
                 tab_v, idx_v, buf0, buf1, out2_v, sem0, sem1):
    wid = lax.axis_index("s") * 2 + lax.axis_index("c")

    pltpu.sync_copy(tab_hbm.at[pl.ds(0, NREL * NH)], tab_v)

    cidx = lax.iota(jnp.int32, 16) + CTAIL
    bufs = (buf0, buf1)
    sems = (sem0, sem1)

    def group(g, carry):
        r0 = (wid * GPW + g) * 8
        pltpu.sync_copy(idx_hbm.at[pl.ds(r0, 8), :], idx_v)
        copies = []
        for hg in range(8):
            p = hg % 2
            if hg >= 2:
                copies[hg - 2].wait()
            buf = bufs[p]
            for rr in range(8):
                def chunk(c, carry2):
                    off = c * 16
                    iv = idx_v[rr, pl.ds(off, 16)]
                    for k in range(2):
                        buf[k, rr, pl.ds(off, 16)] = plsc.load_gather(
                            tab_v, [iv + (hg * 2 + k) * NREL])
                    return carry2

                lax.fori_loop(0, NCH, chunk, 0, unroll=8)
                rsp = jnp.full((16,), rr, jnp.int32)
                iv = plsc.load_gather(idx_v, [rsp, cidx])
                for k in range(2):
                    vals = plsc.load_gather(tab_v,
                                            [iv + (hg * 2 + k) * NREL])
                    plsc.store_scatter(
                        buf, [jnp.full((16,), k, jnp.int32), rsp, cidx],
                        vals)
            copies.append(pltpu.async_copy(
                buf, out_hbm.at[pl.ds(hg * 2, 2), pl.ds(r0, 8), :], sems[p]))
        copies[6].wait()
        copies[7].wait()
        return carry

    lax.fori_loop(0, GPW, group, 0)

    @pl.when(wid == 0)
    def _stray():
        pltpu.sync_copy(idx_hbm.at[pl.ds(1024, 8), :], idx_v)
        rsp0 = jnp.full((16,), 0, jnp.int32)
        iv_t = plsc.load_gather(idx_v, [rsp0, cidx])
        for h in range(NH):
            def chunk(c, carry2):
                off = c * 16
                iv = idx_v[0, pl.ds(off, 16)]
                out2_v[h, pl.ds(off, 16)] = plsc.load_gather(
                    tab_v, [iv + h * NREL])
                return carry2

            lax.fori_loop(0, NCH, chunk, 0, unroll=8)
            vals = plsc.load_gather(tab_v, [iv_t + h * NREL])
            plsc.store_scatter(out2_v, [jnp.full((16,), h, jnp.int32), cidx],
                               vals)
        pltpu.sync_copy(out2_v, out2_hbm.at[pl.ds(0, NH), :])


@jax.jit
def kernel(relative_position_bias_table, relative_position_index):
    tab = relative_position_bias_table.T.reshape(-1)
    idx = jnp.pad(relative_position_index, ((0, 7), (0, 0)))
    out, row_last = _gather_bias(tab, idx)
    out = lax.dynamic_update_slice(
        out, row_last.reshape(NH, 1, L), (0, L - 1, 0))
    return out.reshape(1, NH, L, L)


# speedup vs baseline: 1.2294x; 1.2294x over previous
import functools

import jax
import jax.numpy as jnp
from jax import lax
from jax.experimental import pallas as pl
from jax.experimental.pallas import tpu as pltpu
from jax.experimental.pallas import tpu_sc as plsc

L = 1025
NH = 16
NREL = 3972
NW = 32
GPW = 4
NCH = (L - 1) // 16
CTAIL = L - 16


_mesh = plsc.VectorSubcoreMesh(core_axis_name="c", subcore_axis_name="s")


@functools.partial(
    pl.kernel,
    mesh=_mesh,
    out_type=(
        jax.ShapeDtypeStruct((NH, L, L), jnp.float32),
        jax.ShapeDtypeStruct((NH, L), jnp.float32),
    ),
    scratch_types=[
        pltpu.VMEM((NREL * NH,), jnp.float32),
        pltpu.VMEM((8, L), jnp.int32),
        pltpu.VMEM((4, 8, L), jnp.float32),
        pltpu.VMEM((NH, L), jnp.float32),
    ],
    compiler_params=pltpu.CompilerParams(needs_layout_passes=False),
)
def _gather_bias(tab_hbm, idx_hbm, out_hbm, out2_hbm,
                 tab_v, idx_v, out_v, out2_v):
    wid = lax.axis_index("s") * 2 + lax.axis_index("c")

    pltpu.sync_copy(tab_hbm.at[pl.ds(0, NREL * NH)], tab_v)

    cidx = lax.iota(jnp.int32, 16) + CTAIL

    def group(g, carry):
        r0 = (wid * GPW + g) * 8
        pltpu.sync_copy(idx_hbm.at[pl.ds(r0, 8), :], idx_v)
        for hg in range(4):
            for rr in range(8):
                def chunk(c, carry2):
                    off = c * 16
                    iv = idx_v[rr, pl.ds(off, 16)]
                    for k in range(4):
                        out_v[k, rr, pl.ds(off, 16)] = plsc.load_gather(
                            tab_v, [iv + (hg * 4 + k) * NREL])
                    return carry2

                lax.fori_loop(0, NCH, chunk, 0, unroll=8)
                rsp = jnp.full((16,), rr, jnp.int32)
                iv = plsc.load_gather(idx_v, [rsp, cidx])
                for k in range(4):
                    vals = plsc.load_gather(tab_v,
                                            [iv + (hg * 4 + k) * NREL])
                    plsc.store_scatter(
                        out_v, [jnp.full((16,), k, jnp.int32), rsp, cidx],
                        vals)
            pltpu.sync_copy(out_v,
                            out_hbm.at[pl.ds(hg * 4, 4), pl.ds(r0, 8), :])
        return carry

    lax.fori_loop(0, GPW, group, 0)

    @pl.when(wid == 0)
    def _stray():
        pltpu.sync_copy(idx_hbm.at[pl.ds(1024, 8), :], idx_v)
        rsp0 = jnp.full((16,), 0, jnp.int32)
        iv_t = plsc.load_gather(idx_v, [rsp0, cidx])
        for h in range(NH):
            def chunk(c, carry2):
                off = c * 16
                iv = idx_v[0, pl.ds(off, 16)]
                out2_v[h, pl.ds(off, 16)] = plsc.load_gather(
                    tab_v, [iv + h * NREL])
                return carry2

            lax.fori_loop(0, NCH, chunk, 0, unroll=8)
            vals = plsc.load_gather(tab_v, [iv_t + h * NREL])
            plsc.store_scatter(out2_v, [jnp.full((16,), h, jnp.int32), cidx],
                               vals)
        pltpu.sync_copy(out2_v, out2_hbm.at[pl.ds(0, NH), :])


@jax.jit
def kernel(relative_position_bias_table, relative_position_index):
    tab = relative_position_bias_table.T.reshape(-1)
    idx = jnp.pad(relative_position_index, ((0, 7), (0, 0)))
    out, row_last = _gather_bias(tab, idx)
    out = lax.dynamic_update_slice(
        out, row_last.reshape(NH, 1, L), (0, L - 1, 0))
    return out.reshape(1, NH, L, L)


# speedup vs baseline: 1.2490x; 1.0159x over previous
import functools

import jax
import jax.numpy as jnp
from jax import lax
from jax.experimental import pallas as pl
from jax.experimental.pallas import tpu as pltpu
from jax.experimental.pallas import tpu_sc as plsc

L = 1025
NH = 16
NREL = 3972
NRELP = 3976
NW = 32
GPW = 4
NCH = (L - 1) // 16
CTAIL = L - 16


_mesh = plsc.VectorSubcoreMesh(core_axis_name="c", subcore_axis_name="s")


@functools.partial(
    pl.kernel,
    mesh=_mesh,
    out_type=(
        jax.ShapeDtypeStruct((NH, L, L), jnp.float32),
        jax.ShapeDtypeStruct((NH, L), jnp.float32),
    ),
    scratch_types=[
        pltpu.VMEM((NRELP * NH,), jnp.float32),
        pltpu.VMEM((8, L), jnp.int32),
        pltpu.VMEM((4, 8, L), jnp.float32),
        pltpu.VMEM((NH, L), jnp.float32),
    ],
    compiler_params=pltpu.CompilerParams(needs_layout_passes=False),
)
def _gather_bias(tab_hbm, idx_hbm, out_hbm, out2_hbm,
                 tab_v, idx_v, out_v, out2_v):
    wid = lax.axis_index("s") * 2 + lax.axis_index("c")

    pltpu.sync_copy(tab_hbm.at[pl.ds(0, NRELP * NH)], tab_v)

    cidx = lax.iota(jnp.int32, 16) + CTAIL
    tabs = [tab_v.at[pl.ds(h * NRELP, NRELP)] for h in range(NH)]

    def group(g, carry):
        r0 = (wid * GPW + g) * 8
        pltpu.sync_copy(idx_hbm.at[pl.ds(r0, 8), :], idx_v)
        for hg in range(4):
            for rr in range(8):
                def chunk(c, carry2):
                    off = c * 16
                    iv = idx_v[rr, pl.ds(off, 16)]
                    for k in range(4):
                        out_v[k, rr, pl.ds(off, 16)] = plsc.load_gather(
                            tabs[hg * 4 + k], [iv])
                    return carry2

                lax.fori_loop(0, NCH, chunk, 0, unroll=8)
                rsp = jnp.full((16,), rr, jnp.int32)
                iv = plsc.load_gather(idx_v, [rsp, cidx])
                for k in range(4):
                    vals = plsc.load_gather(tabs[hg * 4 + k], [iv])
                    plsc.store_scatter(
                        out_v, [jnp.full((16,), k, jnp.int32), rsp, cidx],
                        vals)
            pltpu.sync_copy(out_v,
                            out_hbm.at[pl.ds(hg * 4, 4), pl.ds(r0, 8), :])
        return carry

    lax.fori_loop(0, GPW, group, 0)

    @pl.when(wid == 0)
    def _stray():
        pltpu.sync_copy(idx_hbm.at[pl.ds(1024, 8), :], idx_v)
        rsp0 = jnp.full((16,), 0, jnp.int32)
        iv_t = plsc.load_gather(idx_v, [rsp0, cidx])
        for h in range(NH):
            def chunk(c, carry2):
                off = c * 16
                iv = idx_v[0, pl.ds(off, 16)]
                out2_v[h, pl.ds(off, 16)] = plsc.load_gather(
                    tabs[h], [iv])
                return carry2

            lax.fori_loop(0, NCH, chunk, 0, unroll=8)
            vals = plsc.load_gather(tabs[h], [iv_t])
            plsc.store_scatter(out2_v, [jnp.full((16,), h, jnp.int32), cidx],
                               vals)
        pltpu.sync_copy(out2_v, out2_hbm.at[pl.ds(0, NH), :])


@jax.jit
def kernel(relative_position_bias_table, relative_position_index):
    tab = jnp.pad(relative_position_bias_table.T,
                  ((0, 0), (0, NRELP - NREL))).reshape(-1)
    idx = jnp.pad(relative_position_index, ((0, 7), (0, 0)))
    out, row_last = _gather_bias(tab, idx)
    out = lax.dynamic_update_slice(
        out, row_last.reshape(NH, 1, L), (0, L - 1, 0))
    return out.reshape(1, NH, L, L)


# speedup vs baseline: 2.1287x; 1.7043x over previous
import functools

import jax
import jax.numpy as jnp
from jax import lax
from jax.experimental import pallas as pl
from jax.experimental.pallas import tpu as pltpu
from jax.experimental.pallas import tpu_sc as plsc

L = 1025
NH = 16
NREL = 3972
NRELP = 3976
NW = 32
GPW = 4
NCH = (L - 1) // 16
CTAIL = L - 16


_mesh = plsc.VectorSubcoreMesh(core_axis_name="c", subcore_axis_name="s")


@functools.partial(
    pl.kernel,
    mesh=_mesh,
    out_type=(
        jax.ShapeDtypeStruct((NH, L, L), jnp.float32),
        jax.ShapeDtypeStruct((NH, L), jnp.float32),
    ),
    scratch_types=[
        pltpu.VMEM((NRELP * NH,), jnp.float32),
        pltpu.VMEM((8, L), jnp.int32),
        pltpu.VMEM((4, 8, L), jnp.float32),
        pltpu.VMEM((NH, L), jnp.float32),
    ],
    compiler_params=pltpu.CompilerParams(needs_layout_passes=False),
)
def _gather_bias(tab_hbm, idx_hbm, out_hbm, out2_hbm,
                 tab_v, idx_v, out_v, out2_v):
    wid = lax.axis_index("s") * 2 + lax.axis_index("c")

    pltpu.sync_copy(tab_hbm.at[pl.ds(0, NRELP * NH)], tab_v)

    cidx = lax.iota(jnp.int32, 16) + CTAIL
    tabs = [tab_v.at[pl.ds(h * NRELP, NRELP)] for h in range(NH)]

    def group(g, carry):
        r0 = (wid * GPW + g) * 8
        pltpu.sync_copy(idx_hbm.at[pl.ds(r0, 8), :], idx_v)
        for hg in range(4):
            for rr in range(8):
                @plsc.parallel_loop(0, NCH * 16, step=16, unroll=8)
                def chunk(off):
                    iv = idx_v[rr, pl.ds(off, 16)]
                    for k in range(4):
                        out_v[k, rr, pl.ds(off, 16)] = plsc.load_gather(
                            tabs[hg * 4 + k], [iv])
                rsp = jnp.full((16,), rr, jnp.int32)
                iv = plsc.load_gather(idx_v, [rsp, cidx])
                for k in range(4):
                    vals = plsc.load_gather(tabs[hg * 4 + k], [iv])
                    plsc.store_scatter(
                        out_v, [jnp.full((16,), k, jnp.int32), rsp, cidx],
                        vals)
            pltpu.sync_copy(out_v,
                            out_hbm.at[pl.ds(hg * 4, 4), pl.ds(r0, 8), :])
        return carry

    lax.fori_loop(0, GPW, group, 0)

    @pl.when(wid == 0)
    def _stray():
        pltpu.sync_copy(idx_hbm.at[pl.ds(1024, 8), :], idx_v)
        rsp0 = jnp.full((16,), 0, jnp.int32)
        iv_t = plsc.load_gather(idx_v, [rsp0, cidx])
        for h in range(NH):
            @plsc.parallel_loop(0, NCH * 16, step=16, unroll=8)
            def chunk(off):
                iv = idx_v[0, pl.ds(off, 16)]
                out2_v[h, pl.ds(off, 16)] = plsc.load_gather(
                    tabs[h], [iv])
            vals = plsc.load_gather(tabs[h], [iv_t])
            plsc.store_scatter(out2_v, [jnp.full((16,), h, jnp.int32), cidx],
                               vals)
        pltpu.sync_copy(out2_v, out2_hbm.at[pl.ds(0, NH), :])


@jax.jit
def kernel(relative_position_bias_table, relative_position_index):
    tab = jnp.pad(relative_position_bias_table.T,
                  ((0, 0), (0, NRELP - NREL))).reshape(-1)
    idx = jnp.pad(relative_position_index, ((0, 7), (0, 0)))
    out, row_last = _gather_bias(tab, idx)
    out = lax.dynamic_update_slice(
        out, row_last.reshape(NH, 1, L), (0, L - 1, 0))
    return out.reshape(1, NH, L, L)


# speedup vs baseline: 2.3123x; 1.0862x over previous
import functools

import jax
import jax.numpy as jnp
from jax import lax
from jax.experimental import pallas as pl
from jax.experimental.pallas import tpu as pltpu
from jax.experimental.pallas import tpu_sc as plsc

L = 1025
NH = 16
NREL = 3972
NRELP = 3976
NW = 32
GPW = 4
NCH = (L - 1) // 16
CTAIL = L - 16


_mesh = plsc.VectorSubcoreMesh(core_axis_name="c", subcore_axis_name="s")


@functools.partial(
    pl.kernel,
    mesh=_mesh,
    out_type=(
        jax.ShapeDtypeStruct((NH, L, L), jnp.float32),
        jax.ShapeDtypeStruct((NH, L), jnp.float32),
    ),
    scratch_types=[
        pltpu.VMEM((NRELP * NH,), jnp.float32),
        pltpu.VMEM((8, L), jnp.int32),
        pltpu.VMEM((2, 8, L), jnp.float32),
        pltpu.VMEM((2, 8, L), jnp.float32),
        pltpu.VMEM((NH, L), jnp.float32),
        pltpu.SemaphoreType.DMA,
        pltpu.SemaphoreType.DMA,
    ],
    compiler_params=pltpu.CompilerParams(needs_layout_passes=False),
)
def _gather_bias(tab_hbm, idx_hbm, out_hbm, out2_hbm,
                 tab_v, idx_v, buf0, buf1, out2_v, sem0, sem1):
    wid = lax.axis_index("s") * 2 + lax.axis_index("c")

    pltpu.sync_copy(tab_hbm.at[pl.ds(0, NRELP * NH)], tab_v)

    cidx = lax.iota(jnp.int32, 16) + CTAIL
    tabs = [tab_v.at[pl.ds(h * NRELP, NRELP)] for h in range(NH)]
    bufs = (buf0, buf1)
    sems = (sem0, sem1)

    def group(g, carry):
        r0 = (wid * GPW + g) * 8
        pltpu.sync_copy(idx_hbm.at[pl.ds(r0, 8), :], idx_v)
        copies = []
        for hg in range(8):
            p = hg % 2
            if hg >= 2:
                copies[hg - 2].wait()
            buf = bufs[p]
            for rr in range(8):
                @plsc.parallel_loop(0, NCH * 16, step=16, unroll=8)
                def chunk(off):
                    iv = idx_v[rr, pl.ds(off, 16)]
                    for k in range(2):
                        buf[k, rr, pl.ds(off, 16)] = plsc.load_gather(
                            tabs[hg * 2 + k], [iv])
                rsp = jnp.full((16,), rr, jnp.int32)
                iv = plsc.load_gather(idx_v, [rsp, cidx])
                for k in range(2):
                    vals = plsc.load_gather(tabs[hg * 2 + k], [iv])
                    plsc.store_scatter(
                        buf, [jnp.full((16,), k, jnp.int32), rsp, cidx],
                        vals)
            copies.append(pltpu.async_copy(
                buf, out_hbm.at[pl.ds(hg * 2, 2), pl.ds(r0, 8), :], sems[p]))
        copies[6].wait()
        copies[7].wait()
        return carry

    lax.fori_loop(0, GPW, group, 0)

    @pl.when(wid == 0)
    def _stray():
        pltpu.sync_copy(idx_hbm.at[pl.ds(1024, 8), :], idx_v)
        rsp0 = jnp.full((16,), 0, jnp.int32)
        iv_t = plsc.load_gather(idx_v, [rsp0, cidx])
        for h in range(NH):
            @plsc.parallel_loop(0, NCH * 16, step=16, unroll=8)
            def chunk(off):
                iv = idx_v[0, pl.ds(off, 16)]
                out2_v[h, pl.ds(off, 16)] = plsc.load_gather(
                    tabs[h], [iv])
            vals = plsc.load_gather(tabs[h], [iv_t])
            plsc.store_scatter(out2_v, [jnp.full((16,), h, jnp.int32), cidx],
                               vals)
        pltpu.sync_copy(out2_v, out2_hbm.at[pl.ds(0, NH), :])


@jax.jit
def kernel(relative_position_bias_table, relative_position_index):
    tab = jnp.pad(relative_position_bias_table.T,
                  ((0, 0), (0, NRELP - NREL))).reshape(-1)
    idx = jnp.pad(relative_position_index, ((0, 7), (0, 0)))
    out, row_last = _gather_bias(tab, idx)
    out = lax.dynamic_update_slice(
        out, row_last.reshape(NH, 1, L), (0, L - 1, 0))
    return out.reshape(1, NH, L, L)
